# baseline (speedup 1.0000x reference)
import jax
import jax.numpy as jnp
import numpy as np
from jax import lax
from jax.experimental import pallas as pl
from jax.experimental.pallas import tpu as pltpu
from jax.experimental.pallas import tpu_sc as plsc

NUM_FIELDS_RAW = 39
FIELD_DIM = 2560
EMBED_DIM = 16
BATCH = 4096

F = 30
FPAD = 32
TOTAL = F * FIELD_DIM
ROWLEN = F * EMBED_DIM
NC, NS, L = 2, 16, 16
NW = NC * NS
BPW = BATCH // NW
RPT = TOTAL // NW
GRP = 1


def _field_offsets_i32():
    sel = np.full(NUM_FIELDS_RAW, FIELD_DIM, dtype=np.int64)
    sel = np.hstack((sel[:3], sel[4:8], sel[10:15], sel[17:19], sel[21:24], sel[26:]))
    return np.array((0, *np.cumsum(sel)[:-1]), dtype=np.int32)


def _select_cols(x):
    return jnp.concatenate(
        (x[:, :3], x[:, 4:8], x[:, 10:15], x[:, 17:19], x[:, 21:24], x[:, 26:]),
        axis=1)


def _build_kernel(tab_hbm, big_hbm, chunk0, chunk1, sem0, sem1):
    wid = lax.axis_index("s") * NC + lax.axis_index("c")
    r0 = wid * RPT

    chunks = (chunk0, chunk1)
    sems = (sem0, sem1)

    pltpu.async_copy(tab_hbm.at[0, pl.ds(r0, RPT)], chunk0, sem0)
    pltpu.async_copy(tab_hbm.at[1, pl.ds(r0, RPT)], chunk1, sem1)

    def t_body(u, carry):
        t0 = u * 2
        for k in range(2):
            t = t0 + k
            ch, sm = chunks[k], sems[k]
            pltpu.make_async_copy(tab_hbm.at[t, pl.ds(r0, RPT)], ch, sm).wait()
            pltpu.sync_copy(
                ch,
                big_hbm.at[pl.ds(r0, RPT),
                           pl.ds(pl.multiple_of(t * EMBED_DIM, L), L)])

            @pl.when(t + 2 < F)
            def _():
                pltpu.async_copy(tab_hbm.at[t + 2, pl.ds(r0, RPT)], ch, sm)

        return carry

    lax.fori_loop(0, F // 2, t_body, 0, unroll=False)


def _ffm_kernel(big_hbm, w_hbm, xi_hbm, bias_hbm, out_hbm,
                xi_v, w_all_v, idx0, rows0,
                z_v, bias_v, sem0):
    idx_bufs = (idx0,)
    rows_bufs = (rows0,)
    sems = (sem0,)
    wid = lax.axis_index("s") * NC + lax.axis_index("c")
    base = wid * BPW
    pltpu.sync_copy(xi_hbm.at[pl.ds(base, BPW)], xi_v)
    pltpu.sync_copy(bias_hbm, bias_v)
    pltpu.sync_copy(w_hbm, w_all_v)

    lanes = lax.iota(jnp.int32, L)
    pa = jnp.where(lanes < 2, lanes + 14, 0)
    pb = jnp.where(lanes < 2, 0, lanes - 2)

    def build_idx(s, idx_v):
        xa = xi_v[s, pl.ds(0, L)]
        xb = xi_v[s, pl.ds(L, L)]
        idx_v[pl.ds(0, L)] = xa
        tail = jnp.where(
            lanes < 2,
            jnp.take_along_axis(xa, pa, axis=0, mode="promise_in_bounds"),
            jnp.take_along_axis(xb, pb, axis=0, mode="promise_in_bounds"))
        idx_v[pl.ds(F - L, L)] = tail

    def group_body(g, zvec):
        s0 = g * GRP
        handles = []
        for k in range(GRP):
            build_idx(s0 + k, idx_bufs[k])
            handles.append(
                pltpu.async_copy(big_hbm.at[idx_bufs[k]], rows_bufs[k], sems[k]))
        for k in range(GRP):
            handles[k].wait()
            rows_v = rows_bufs[k]
            xa = xi_v[s0 + k, pl.ds(0, L)]
            xb = xi_v[s0 + k, pl.ds(L, L)]
            ga = plsc.load_gather(w_all_v, [xa])
            gb = plsc.load_gather(w_all_v, [xb])
            acc = ga + jnp.where(lanes < F - L, gb, 0.0)
            for i in range(F):
                for j in range(i + 1, F):
                    acc = acc + (rows_v[j, pl.ds(EMBED_DIM * i, L)]
                                 * rows_v[i, pl.ds(EMBED_DIM * j, L)])
            for sh in (1, 2, 4, 8):
                acc = acc + jnp.take_along_axis(
                    acc, lanes ^ sh, axis=0, mode="promise_in_bounds")
            lane = (s0 + k) % L
            zvec = jnp.where(lanes == lane, acc, zvec)

            @pl.when(lane == L - 1)
            def _():
                z_v[pl.ds(pl.multiple_of(((s0 + k) // L) * L, L), L)] = zvec

        return zvec

    lax.fori_loop(0, BPW // GRP, group_body, jnp.zeros((L,), jnp.float32))

    for g in range(BPW // L):
        zz = z_v[pl.ds(g * L, L)]
        z_v[pl.ds(g * L, L)] = 1.0 / (1.0 + jnp.exp(-(zz + bias_v[...])))
    pltpu.sync_copy(z_v, out_hbm.at[pl.ds(base, BPW)])


@jax.jit
def _run(tab, w1d, xi_pad, bias16):
    mesh = plsc.VectorSubcoreMesh(
        core_axis_name="c", subcore_axis_name="s", num_cores=NC, num_subcores=NS)
    big = pl.kernel(
        _build_kernel,
        out_type=jax.ShapeDtypeStruct((TOTAL, ROWLEN), jnp.float32),
        mesh=mesh,
        compiler_params=pltpu.CompilerParams(use_tc_tiling_on_sc=False),
        scratch_types=[
            pltpu.VMEM((RPT, EMBED_DIM), jnp.float32),
            pltpu.VMEM((RPT, EMBED_DIM), jnp.float32),
            pltpu.SemaphoreType.DMA,
            pltpu.SemaphoreType.DMA,
        ],
    )(tab)
    mesh2 = plsc.VectorSubcoreMesh(
        core_axis_name="c", subcore_axis_name="s", num_cores=NC, num_subcores=NS)
    return pl.kernel(
        _ffm_kernel,
        out_type=jax.ShapeDtypeStruct((BATCH,), jnp.float32),
        mesh=mesh2,
        compiler_params=pltpu.CompilerParams(
            use_tc_tiling_on_sc=False, needs_layout_passes=False),
        scratch_types=[
            pltpu.VMEM((BPW, FPAD), jnp.int32),
            pltpu.VMEM((TOTAL,), jnp.float32),
            pltpu.VMEM((F,), jnp.int32),
            pltpu.VMEM((F, ROWLEN), jnp.float32),
            pltpu.VMEM((BPW,), jnp.float32),
            pltpu.VMEM((L,), jnp.float32),
            pltpu.SemaphoreType.DMA,
        ],
    )(big, w1d, xi_pad, bias16)


def kernel(x, additional, W_lin, bias, ffm_tables):
    offsets = jnp.asarray(_field_offsets_i32())
    xi = _select_cols(x).astype(jnp.int32) + offsets[None, :]
    xi_pad = jnp.pad(xi, ((0, 0), (0, FPAD - F)))
    bias16 = jnp.broadcast_to(bias.astype(jnp.float32), (L,))
    return _run(ffm_tables, W_lin.astype(jnp.float32).reshape(TOTAL),
                xi_pad, bias16)


# speedup vs baseline: 2.2232x; 2.2232x over previous
import jax
import jax.numpy as jnp
import numpy as np
from jax import lax
from jax.experimental import pallas as pl
from jax.experimental.pallas import tpu as pltpu
from jax.experimental.pallas import tpu_sc as plsc

NUM_FIELDS_RAW = 39
FIELD_DIM = 2560
EMBED_DIM = 16
BATCH = 4096

F = 30
FPAD = 32
TOTAL = F * FIELD_DIM
ROWLEN = F * EMBED_DIM
NC, NS, L = 2, 16, 16
NW = NC * NS
BPW = BATCH // NW


def _field_offsets_i32():
    sel = np.full(NUM_FIELDS_RAW, FIELD_DIM, dtype=np.int64)
    sel = np.hstack((sel[:3], sel[4:8], sel[10:15], sel[17:19], sel[21:24], sel[26:]))
    return np.array((0, *np.cumsum(sel)[:-1]), dtype=np.int32)


def _select_cols(x):
    return jnp.concatenate(
        (x[:, :3], x[:, 4:8], x[:, 10:15], x[:, 17:19], x[:, 21:24], x[:, 26:]),
        axis=1)


def _ffm_kernel(big_hbm, w_hbm, xi_hbm, bias_hbm, out_hbm,
                xi_v, w_all_v, idx_v, rows_v, z_v, bias_v, sem):
    wid = lax.axis_index("s") * NC + lax.axis_index("c")
    base = wid * BPW
    pltpu.sync_copy(xi_hbm.at[pl.ds(base, BPW)], xi_v)
    pltpu.sync_copy(bias_hbm, bias_v)
    pltpu.sync_copy(w_hbm, w_all_v)

    lanes = lax.iota(jnp.int32, L)
    pa = jnp.where(lanes < 2, lanes + 14, 0)
    pb = jnp.where(lanes < 2, 0, lanes - 2)

    def sample_body(s, zvec):
        xa = xi_v[s, pl.ds(0, L)]
        xb = xi_v[s, pl.ds(L, L)]
        idx_v[pl.ds(0, L)] = xa
        tail = jnp.where(
            lanes < 2,
            jnp.take_along_axis(xa, pa, axis=0, mode="promise_in_bounds"),
            jnp.take_along_axis(xb, pb, axis=0, mode="promise_in_bounds"))
        idx_v[pl.ds(F - L, L)] = tail
        h = pltpu.async_copy(big_hbm.at[idx_v], rows_v, sem)

        ga = plsc.load_gather(w_all_v, [xa])
        gb = plsc.load_gather(w_all_v, [xb])
        acc = ga + jnp.where(lanes < F - L, gb, 0.0)

        h.wait()
        for i in range(F):
            for j in range(i + 1, F):
                acc = acc + (rows_v[j, pl.ds(EMBED_DIM * i, L)]
                             * rows_v[i, pl.ds(EMBED_DIM * j, L)])
        for sh in (1, 2, 4, 8):
            acc = acc + jnp.take_along_axis(
                acc, lanes ^ sh, axis=0, mode="promise_in_bounds")
        lane = s % L
        zvec = jnp.where(lanes == lane, acc, zvec)

        @pl.when(lane == L - 1)
        def _():
            z_v[pl.ds(pl.multiple_of((s // L) * L, L), L)] = zvec

        return zvec

    lax.fori_loop(0, BPW, sample_body, jnp.zeros((L,), jnp.float32))

    for g in range(BPW // L):
        zz = z_v[pl.ds(g * L, L)]
        z_v[pl.ds(g * L, L)] = 1.0 / (1.0 + jnp.exp(-(zz + bias_v[...])))
    pltpu.sync_copy(z_v, out_hbm.at[pl.ds(base, BPW)])


@jax.jit
def _run(big, w1d, xi_pad, bias16):
    mesh = plsc.VectorSubcoreMesh(
        core_axis_name="c", subcore_axis_name="s", num_cores=NC, num_subcores=NS)
    return pl.kernel(
        _ffm_kernel,
        out_type=jax.ShapeDtypeStruct((BATCH,), jnp.float32),
        mesh=mesh,
        compiler_params=pltpu.CompilerParams(
            use_tc_tiling_on_sc=False, needs_layout_passes=False),
        scratch_types=[
            pltpu.VMEM((BPW, FPAD), jnp.int32),
            pltpu.VMEM((TOTAL,), jnp.float32),
            pltpu.VMEM((F,), jnp.int32),
            pltpu.VMEM((F, ROWLEN), jnp.float32),
            pltpu.VMEM((BPW,), jnp.float32),
            pltpu.VMEM((L,), jnp.float32),
            pltpu.SemaphoreType.DMA,
        ],
    )(big, w1d, xi_pad, bias16)


def kernel(x, additional, W_lin, bias, ffm_tables):
    offsets = jnp.asarray(_field_offsets_i32())
    xi = _select_cols(x).astype(jnp.int32) + offsets[None, :]
    xi_pad = jnp.pad(xi, ((0, 0), (0, FPAD - F)))
    big = jnp.swapaxes(ffm_tables, 0, 1).reshape(TOTAL, ROWLEN)
    bias16 = jnp.broadcast_to(bias.astype(jnp.float32), (L,))
    return _run(big, W_lin.astype(jnp.float32).reshape(TOTAL), xi_pad, bias16)


# speedup vs baseline: 2.6241x; 1.1803x over previous
import jax
import jax.numpy as jnp
import numpy as np
from jax import lax
from jax.experimental import pallas as pl
from jax.experimental.pallas import tpu as pltpu
from jax.experimental.pallas import tpu_sc as plsc

NUM_FIELDS_RAW = 39
FIELD_DIM = 2560
EMBED_DIM = 16
BATCH = 4096

F = 30
FPAD = 32
TOTAL = F * FIELD_DIM
ROWLEN = 512
NC, NS, L = 2, 16, 16
NW = NC * NS
BPW = BATCH // NW


def _field_offsets_i32():
    sel = np.full(NUM_FIELDS_RAW, FIELD_DIM, dtype=np.int64)
    sel = np.hstack((sel[:3], sel[4:8], sel[10:15], sel[17:19], sel[21:24], sel[26:]))
    return np.array((0, *np.cumsum(sel)[:-1]), dtype=np.int32)


def _select_cols(x):
    return jnp.concatenate(
        (x[:, :3], x[:, 4:8], x[:, 10:15], x[:, 17:19], x[:, 21:24], x[:, 26:]),
        axis=1)


def _ffm_kernel(big_hbm, w_hbm, xi_hbm, bias_hbm, out_hbm,
                xi_v, w_all_v, idx_v, rows_v, z_v, bias_v, sem):
    wid = lax.axis_index("s") * NC + lax.axis_index("c")
    base = wid * BPW
    pltpu.sync_copy(xi_hbm.at[pl.ds(base, BPW)], xi_v)
    pltpu.sync_copy(bias_hbm, bias_v)
    pltpu.sync_copy(w_hbm, w_all_v)

    lanes = lax.iota(jnp.int32, L)
    pa = jnp.where(lanes < 2, lanes + 14, 0)
    pb = jnp.where(lanes < 2, 0, lanes - 2)

    def sample_body(s, zvec):
        xa = xi_v[s, pl.ds(0, L)]
        xb = xi_v[s, pl.ds(L, L)]
        idx_v[pl.ds(0, L)] = xa
        tail = jnp.where(
            lanes < 2,
            jnp.take_along_axis(xa, pa, axis=0, mode="promise_in_bounds"),
            jnp.take_along_axis(xb, pb, axis=0, mode="promise_in_bounds"))
        idx_v[pl.ds(F - L, L)] = tail
        h = pltpu.async_copy(big_hbm.at[idx_v], rows_v, sem)

        ga = plsc.load_gather(w_all_v, [xa])
        gb = plsc.load_gather(w_all_v, [xb])
        acc = ga + jnp.where(lanes < F - L, gb, 0.0)

        h.wait()
        for i in range(F):
            for j in range(i + 1, F):
                acc = acc + (rows_v[j, pl.ds(EMBED_DIM * i, L)]
                             * rows_v[i, pl.ds(EMBED_DIM * j, L)])
        for sh in (1, 2, 4, 8):
            acc = acc + jnp.take_along_axis(
                acc, lanes ^ sh, axis=0, mode="promise_in_bounds")
        lane = s % L
        zvec = jnp.where(lanes == lane, acc, zvec)

        @pl.when(lane == L - 1)
        def _():
            z_v[pl.ds(pl.multiple_of((s // L) * L, L), L)] = zvec

        return zvec

    lax.fori_loop(0, BPW, sample_body, jnp.zeros((L,), jnp.float32))

    for g in range(BPW // L):
        zz = z_v[pl.ds(g * L, L)]
        z_v[pl.ds(g * L, L)] = 1.0 / (1.0 + jnp.exp(-(zz + bias_v[...])))
    pltpu.sync_copy(z_v, out_hbm.at[pl.ds(base, BPW)])


@jax.jit
def _run(big, w1d, xi_pad, bias16):
    mesh = plsc.VectorSubcoreMesh(
        core_axis_name="c", subcore_axis_name="s", num_cores=NC, num_subcores=NS)
    return pl.kernel(
        _ffm_kernel,
        out_type=jax.ShapeDtypeStruct((BATCH,), jnp.float32),
        mesh=mesh,
        compiler_params=pltpu.CompilerParams(
            use_tc_tiling_on_sc=True, needs_layout_passes=False),
        scratch_types=[
            pltpu.VMEM((BPW, FPAD), jnp.int32),
            pltpu.VMEM((TOTAL,), jnp.float32),
            pltpu.VMEM((F,), jnp.int32),
            pltpu.VMEM((F, ROWLEN), jnp.float32),
            pltpu.VMEM((BPW,), jnp.float32),
            pltpu.VMEM((L,), jnp.float32),
            pltpu.SemaphoreType.DMA,
        ],
    )(big, w1d, xi_pad, bias16)


def kernel(x, additional, W_lin, bias, ffm_tables):
    offsets = jnp.asarray(_field_offsets_i32())
    xi = _select_cols(x).astype(jnp.int32) + offsets[None, :]
    xi_pad = jnp.pad(xi, ((0, 0), (0, FPAD - F)))
    big = jnp.pad(
        jnp.swapaxes(ffm_tables, 0, 1).reshape(TOTAL, F * EMBED_DIM),
        ((0, 0), (0, ROWLEN - F * EMBED_DIM)))
    bias16 = jnp.broadcast_to(bias.astype(jnp.float32), (L,))
    return _run(big, W_lin.astype(jnp.float32).reshape(TOTAL), xi_pad, bias16)
